# baseline (speedup 1.0000x reference)
import functools

import jax
import jax.numpy as jnp
from jax import lax
from jax.experimental import pallas as pl
from jax.experimental.pallas import tpu as pltpu
from jax.experimental.pallas import tpu_sc as plsc


def _soft_embedding_call(tokens_flat, wte_weight, learned_embedding, B, S, D, NT):
    NT8 = (NT // 8) * 8
    G = S - NT8
    info = plsc.get_sparse_core_info()
    NC, NS = info.num_cores, info.num_subcores
    NW = NC * NS
    RPW = B // NW
    C1 = min(G, 128)
    C2 = G - C1

    mesh = plsc.VectorSubcoreMesh(core_axis_name="c", subcore_axis_name="s")

    @functools.partial(
        pl.kernel,
        mesh=mesh,
        out_type=jax.ShapeDtypeStruct((B, S, D), jnp.float32),
        compiler_params=pltpu.CompilerParams(use_tc_tiling_on_sc=False),
        scratch_types=[
            pltpu.VMEM((RPW * S,), jnp.int32),
            pltpu.VMEM((S, D), jnp.float32),
            pltpu.VMEM((NT, D), jnp.float32),
            pltpu.SemaphoreType.DMA,
        ],
    )
    def soft_embed(tok_hbm, wte_hbm, le_hbm, out_hbm, toks_v, rows_v, le_v, sem):
        wid = lax.axis_index("s") * NC + lax.axis_index("c")
        base = wid * RPW
        pltpu.sync_copy(tok_hbm.at[pl.ds(base * S, RPW * S)], toks_v)
        pltpu.sync_copy(le_hbm, le_v)
        pltpu.sync_copy(le_hbm, rows_v.at[pl.ds(0, NT)])

        def body(i, carry):
            off = i * S + NT8
            g1 = pltpu.async_copy(
                wte_hbm.at[toks_v.at[pl.ds(off, C1)]],
                rows_v.at[pl.ds(NT8, C1)], sem)
            g2 = pltpu.async_copy(
                wte_hbm.at[toks_v.at[pl.ds(off + C1, C2)]],
                rows_v.at[pl.ds(NT8 + C1, C2)], sem)
            g1.wait()
            g2.wait()
            for r in range(NT8, NT):
                for c in range(0, D, 16):
                    rows_v[r, pl.ds(c, 16)] = le_v[r, pl.ds(c, 16)]
            pltpu.sync_copy(rows_v, out_hbm.at[base + i])
            return carry

        lax.fori_loop(0, RPW, body, 0)

    return soft_embed(tokens_flat, wte_weight, learned_embedding)


def kernel(tokens, wte_weight, learned_embedding):
    B, S = tokens.shape
    V, D = wte_weight.shape
    NT = learned_embedding.shape[0]
    tokens_flat = tokens.reshape(-1).astype(jnp.int32)
    return _soft_embedding_call(
        tokens_flat, wte_weight, learned_embedding, B, S, D, NT)


# speedup vs baseline: 1.0303x; 1.0303x over previous
import functools

import jax
import jax.numpy as jnp
from jax import lax
from jax.experimental import pallas as pl
from jax.experimental.pallas import tpu as pltpu
from jax.experimental.pallas import tpu_sc as plsc

_NBUF = 4


def _soft_embedding_call(tokens_flat, wte_weight, learned_embedding, B, S, D, NT):
    NT8 = (NT // 8) * 8
    G = S - NT8
    GO = NT - NT8
    GR = S - NT
    info = plsc.get_sparse_core_info()
    NC, NS = info.num_cores, info.num_subcores
    NW = NC * NS
    RPW = B // NW
    C1 = min(G, 128)
    C2 = G - C1

    mesh = plsc.VectorSubcoreMesh(core_axis_name="c", subcore_axis_name="s")

    @functools.partial(
        pl.kernel,
        mesh=mesh,
        out_type=jax.ShapeDtypeStruct((B, S, D), jnp.float32),
        compiler_params=pltpu.CompilerParams(use_tc_tiling_on_sc=False),
        scratch_types=[
            pltpu.VMEM((RPW * S,), jnp.int32),
            pltpu.VMEM((_NBUF, G, D), jnp.float32),
            pltpu.VMEM((NT, D), jnp.float32),
            pltpu.SemaphoreType.DMA((_NBUF,)),
            pltpu.SemaphoreType.DMA((_NBUF,)),
        ],
    )
    def soft_embed(tok_hbm, wte_hbm, le_hbm, out_hbm, toks_v, gath_v, le_v,
                   gsem, wsem):
        wid = lax.axis_index("s") * NC + lax.axis_index("c")
        base = wid * RPW
        pltpu.sync_copy(tok_hbm.at[pl.ds(base * S, RPW * S)], toks_v)
        pltpu.sync_copy(le_hbm, le_v)

        def gather_copies(g, b):
            off = pl.multiple_of(g * S + NT8, 8)
            cs = [pltpu.make_async_copy(
                wte_hbm.at[toks_v.at[pl.ds(off, C1)]],
                gath_v.at[b, pl.ds(0, C1)], gsem.at[b])]
            if C2:
                cs.append(pltpu.make_async_copy(
                    wte_hbm.at[toks_v.at[pl.ds(off + C1, C2)]],
                    gath_v.at[b, pl.ds(C1, C2)], gsem.at[b]))
            return cs

        def write_copies(g, b):
            row = base + g
            return [
                pltpu.make_async_copy(
                    le_v, out_hbm.at[row, pl.ds(0, NT)], wsem.at[b]),
                pltpu.make_async_copy(
                    gath_v.at[b, pl.ds(GO, GR)],
                    out_hbm.at[row, pl.ds(NT, GR)], wsem.at[b]),
            ]

        def start(cs):
            for c in cs:
                c.start()

        def wait(cs):
            for c in cs:
                c.wait()

        for b in range(_NBUF):
            start(gather_copies(b, b))

        def outer(k, carry):
            g0 = k * _NBUF
            for b in range(_NBUF):
                g = g0 + b
                wait(gather_copies(g, b))
                start(write_copies(g, b))
                wait(write_copies(g, b))
                start(gather_copies(g + _NBUF, b))
            return carry

        lax.fori_loop(0, RPW // _NBUF - 1, outer, 0)

        for b in range(_NBUF):
            g = RPW - _NBUF + b
            wait(gather_copies(g, b))
            start(write_copies(g, b))
        for b in range(_NBUF):
            wait(write_copies(RPW - _NBUF + b, b))

    return soft_embed(tokens_flat, wte_weight, learned_embedding)


def kernel(tokens, wte_weight, learned_embedding):
    B, S = tokens.shape
    V, D = wte_weight.shape
    NT = learned_embedding.shape[0]
    tokens_flat = tokens.reshape(-1).astype(jnp.int32)
    return _soft_embedding_call(
        tokens_flat, wte_weight, learned_embedding, B, S, D, NT)


# speedup vs baseline: 1.0327x; 1.0024x over previous
import functools

import jax
import jax.numpy as jnp
from jax import lax
from jax.experimental import pallas as pl
from jax.experimental.pallas import tpu as pltpu
from jax.experimental.pallas import tpu_sc as plsc

_NBUF = 4


def _soft_embedding_call(tokens, wte_weight, learned_embedding, B, S, D, NT):
    NT8 = (NT // 8) * 8
    G = S - NT8
    GO = NT - NT8
    GR = S - NT
    info = plsc.get_sparse_core_info()
    NC, NS = info.num_cores, info.num_subcores
    NW = NC * NS
    RPW = B // NW
    C1 = min(G, 128)
    C2 = G - C1

    mesh = plsc.VectorSubcoreMesh(core_axis_name="c", subcore_axis_name="s")

    @functools.partial(
        pl.kernel,
        mesh=mesh,
        out_type=jax.ShapeDtypeStruct((B, S, D), jnp.float32),
        compiler_params=pltpu.CompilerParams(use_tc_tiling_on_sc=False),
        scratch_types=[
            pltpu.VMEM((RPW, S), jnp.int32),
            pltpu.VMEM((_NBUF, G, D), jnp.float32),
            pltpu.VMEM((NT, D), jnp.float32),
            pltpu.SemaphoreType.DMA((_NBUF,)),
            pltpu.SemaphoreType.DMA((_NBUF,)),
        ],
    )
    def soft_embed(tok_hbm, wte_hbm, le_hbm, out_hbm, toks_v, gath_v, le_v,
                   gsem, wsem):
        wid = lax.axis_index("s") * NC + lax.axis_index("c")
        base = wid * RPW
        pltpu.sync_copy(tok_hbm.at[pl.ds(base, RPW)], toks_v)
        pltpu.sync_copy(le_hbm, le_v)

        def gather_copies(g, b):
            cs = [pltpu.make_async_copy(
                wte_hbm.at[toks_v.at[g, pl.ds(NT8, C1)]],
                gath_v.at[b, pl.ds(0, C1)], gsem.at[b])]
            if C2:
                cs.append(pltpu.make_async_copy(
                    wte_hbm.at[toks_v.at[g, pl.ds(NT8 + C1, C2)]],
                    gath_v.at[b, pl.ds(C1, C2)], gsem.at[b]))
            return cs

        def write_copies(g, b):
            row = base + g
            return [
                pltpu.make_async_copy(
                    le_v, out_hbm.at[row, pl.ds(0, NT)], wsem.at[b]),
                pltpu.make_async_copy(
                    gath_v.at[b, pl.ds(GO, GR)],
                    out_hbm.at[row, pl.ds(NT, GR)], wsem.at[b]),
            ]

        def start(cs):
            for c in cs:
                c.start()

        def wait(cs):
            for c in cs:
                c.wait()

        for b in range(_NBUF):
            start(gather_copies(b, b))

        def outer(k, carry):
            g0 = k * _NBUF
            for b in range(_NBUF):
                g = g0 + b
                wait(gather_copies(g, b))
                start(write_copies(g, b))
                wait(write_copies(g, b))
                start(gather_copies(g + _NBUF, b))
            return carry

        lax.fori_loop(0, RPW // _NBUF - 1, outer, 0)

        for b in range(_NBUF):
            g = RPW - _NBUF + b
            wait(gather_copies(g, b))
            start(write_copies(g, b))
        for b in range(_NBUF):
            wait(write_copies(RPW - _NBUF + b, b))

    return soft_embed(tokens, wte_weight, learned_embedding)


def kernel(tokens, wte_weight, learned_embedding):
    B, S = tokens.shape
    V, D = wte_weight.shape
    NT = learned_embedding.shape[0]
    tokens = tokens.astype(jnp.int32)
    return _soft_embedding_call(
        tokens, wte_weight, learned_embedding, B, S, D, NT)
